# baseline (speedup 1.0000x reference)
import functools

import jax
import jax.numpy as jnp
from jax import lax
from jax.experimental import pallas as pl
from jax.experimental.pallas import tpu as pltpu
from jax.experimental.pallas import tpu_sc as plsc

_NC = 2
_NS = 16
_NW = _NC * _NS

_KI = 8
_CHUNK = _KI * 128
_LANES = 16


def _make_fused(num_rows: int, embed_dim: int, feat: int):
    assert num_rows % (_NW * _CHUNK) == 0
    out_dim = embed_dim + feat - 1
    rows_per_w = num_rows // _NW
    steps = rows_per_w // _CHUNK
    assert steps >= 2 and steps % 2 == 0

    mesh = plsc.VectorSubcoreMesh(core_axis_name="c", subcore_axis_name="s")

    @functools.partial(
        pl.kernel,
        mesh=mesh,
        out_type=jax.ShapeDtypeStruct((num_rows, out_dim), jnp.float32),
        scratch_types=[
            pltpu.VMEM((_KI, 128), jnp.int32),
            pltpu.VMEM((_KI, 128), jnp.int32),
            pltpu.VMEM((_KI, 128), jnp.int32),
            pltpu.VMEM((_KI, 128), jnp.int32),
            pltpu.VMEM((_KI, 128), jnp.float32),
            pltpu.VMEM((_KI, 128), jnp.float32),
            pltpu.VMEM((_CHUNK, 32), jnp.float32),
            pltpu.VMEM((_CHUNK, 32), jnp.float32),
            pltpu.SemaphoreType.DMA,
            pltpu.SemaphoreType.DMA,
            pltpu.SemaphoreType.DMA,
            pltpu.SemaphoreType.DMA,
            pltpu.SemaphoreType.DMA,
            pltpu.SemaphoreType.DMA,
            pltpu.SemaphoreType.DMA,
        ],
        compiler_params=pltpu.CompilerParams(use_tc_tiling_on_sc=False),
    )
    def fused_kernel(
        x1_hbm, feats_hbm, table_hbm, out_hbm,
        idx0, idx1, ixx0, ixx1, idf0, idf1, emb0, emb1,
        f0, f1, g0, g1, s0, s1, fx,
    ):
        wid = lax.axis_index("s") * _NC + lax.axis_index("c")
        row_base = wid * rows_per_w
        idx_v = (idx0, idx1)
        ixx_v = (ixx0, ixx1)
        idf_v = (idf0, idf1)
        emb_v = (emb0, emb1)
        fsem = (f0, f1)
        gsem = (g0, g1)
        ssem = (s0, s1)

        def feats_giant_copy():
            return pltpu.make_async_copy(
                feats_hbm.at[pl.ds(row_base, rows_per_w)],
                out_hbm.at[pl.ds(row_base, rows_per_w), pl.ds(embed_dim, feat - 1)],
                fx,
            )

        def id_stage(i, b):
            return [
                pltpu.make_async_copy(
                    x1_hbm.at[ixx_v[b].at[j]],
                    idf_v[b].at[j],
                    fsem[b],
                )
                for j in range(_KI)
            ]

        def build_id_indices(i, b):
            r0 = row_base + i * _CHUNK
            iota = lax.iota(jnp.int32, _LANES)
            for j in range(_KI):
                for c in range(128 // _LANES):
                    base = r0 + j * 128 + c * _LANES
                    ixx_v[b][j, pl.ds(c * _LANES, _LANES)] = (iota + base) * feat

        def convert_ids(b):
            for j in range(_KI):
                for c in range(128 // _LANES):
                    v = idf_v[b][j, pl.ds(c * _LANES, _LANES)]
                    idx_v[b][j, pl.ds(c * _LANES, _LANES)] = v.astype(jnp.int32)

        def issue_gathers(b):
            return [
                pltpu.async_copy(
                    table_hbm.at[idx_v[b].at[j]],
                    emb_v[b].at[pl.ds(j * 128, 128)],
                    gsem[b],
                )
                for j in range(_KI)
            ]

        def store_copy(i, b):
            return pltpu.make_async_copy(
                emb_v[b],
                out_hbm.at[pl.ds(row_base + i * _CHUNK, _CHUNK), pl.ds(0, embed_dim)],
                ssem[b],
            )

        def run_chunk(i, b, first, last):
            for d in id_stage(i, b):
                d.wait()
            convert_ids(b)
            gathers = issue_gathers(b)
            if not first:
                store_copy(i - 1, 1 - b).wait()
            if not last:
                build_id_indices(i + 1, 1 - b)
                for d in id_stage(i + 1, 1 - b):
                    d.start()
            for g in gathers:
                g.wait()
            store_copy(i, b).start()

        feats_giant_copy().start()
        build_id_indices(0, 0)
        for d in id_stage(0, 0):
            d.start()
        run_chunk(0, 0, first=True, last=False)

        def pair(g, carry):
            i1 = 1 + 2 * g
            run_chunk(i1, 1, first=False, last=False)
            run_chunk(i1 + 1, 0, first=False, last=False)
            return carry

        lax.fori_loop(0, (steps - 2) // 2, pair, 0)

        run_chunk(steps - 1, 1, first=False, last=True)
        store_copy(steps - 1, 1).wait()
        feats_giant_copy().wait()

    return fused_kernel


def kernel(x, embed_weight):
    batch, seq, feat = x.shape
    num_rows = batch * seq
    embed_dim = embed_weight.shape[1]

    x2 = x.reshape(num_rows, feat)
    x1 = x.reshape(num_rows * feat)
    feats = x2[:, 1:]
    out = _make_fused(num_rows, embed_dim, feat)(x1, feats, embed_weight)
    return out.reshape(batch, seq, embed_dim + feat - 1)


# speedup vs baseline: 1.0383x; 1.0383x over previous
import functools

import jax
import jax.numpy as jnp
from jax import lax
from jax.experimental import pallas as pl
from jax.experimental.pallas import tpu as pltpu
from jax.experimental.pallas import tpu_sc as plsc

_NC = 2
_NS = 16
_NW = _NC * _NS

_KI = 8
_CHUNK = _KI * 128


def _make_fused(num_rows: int, embed_dim: int, feat: int):
    assert num_rows % (_NW * _CHUNK) == 0
    out_dim = embed_dim + feat - 1
    rows_per_w = num_rows // _NW
    steps = rows_per_w // _CHUNK
    irows_per_w = rows_per_w // 128
    assert steps >= 2 and steps % 2 == 0

    mesh = plsc.VectorSubcoreMesh(core_axis_name="c", subcore_axis_name="s")

    @functools.partial(
        pl.kernel,
        mesh=mesh,
        out_type=jax.ShapeDtypeStruct((num_rows, out_dim), jnp.float32),
        scratch_types=[
            pltpu.VMEM((_KI, 128), jnp.int32),
            pltpu.VMEM((_KI, 128), jnp.int32),
            pltpu.VMEM((_CHUNK, 32), jnp.float32),
            pltpu.VMEM((_CHUNK, 32), jnp.float32),
            pltpu.SemaphoreType.DMA,
            pltpu.SemaphoreType.DMA,
            pltpu.SemaphoreType.DMA,
            pltpu.SemaphoreType.DMA,
            pltpu.SemaphoreType.DMA,
            pltpu.SemaphoreType.DMA,
            pltpu.SemaphoreType.DMA,
        ],
        compiler_params=pltpu.CompilerParams(use_tc_tiling_on_sc=False),
    )
    def fused_kernel(
        ids_hbm, feats_hbm, table_hbm, out_hbm,
        idx0, idx1, emb0, emb1,
        f0, f1, g0, g1, s0, s1, fx,
    ):
        wid = lax.axis_index("s") * _NC + lax.axis_index("c")
        row_base = wid * rows_per_w
        irow_base = wid * irows_per_w
        idx_v = (idx0, idx1)
        emb_v = (emb0, emb1)
        fsem = (f0, f1)
        gsem = (g0, g1)
        ssem = (s0, s1)

        def feats_giant_copy():
            return pltpu.make_async_copy(
                feats_hbm.at[pl.ds(row_base, rows_per_w)],
                out_hbm.at[pl.ds(row_base, rows_per_w), pl.ds(embed_dim, feat - 1)],
                fx,
            )

        def idx_copy(i, b):
            return pltpu.make_async_copy(
                ids_hbm.at[pl.ds(irow_base + i * _KI, _KI)],
                idx_v[b],
                fsem[b],
            )

        def issue_gathers(b):
            return [
                pltpu.async_copy(
                    table_hbm.at[idx_v[b].at[j]],
                    emb_v[b].at[pl.ds(j * 128, 128)],
                    gsem[b],
                )
                for j in range(_KI)
            ]

        def store_copy(i, b):
            return pltpu.make_async_copy(
                emb_v[b],
                out_hbm.at[pl.ds(row_base + i * _CHUNK, _CHUNK), pl.ds(0, embed_dim)],
                ssem[b],
            )

        def run_chunk(i, b, first, last):
            idx_copy(i, b).wait()
            gathers = issue_gathers(b)
            if not first:
                store_copy(i - 1, 1 - b).wait()
            if not last:
                idx_copy(i + 1, 1 - b).start()
            for g in gathers:
                g.wait()
            store_copy(i, b).start()

        feats_giant_copy().start()
        idx_copy(0, 0).start()
        run_chunk(0, 0, first=True, last=False)

        def pair(g, carry):
            i1 = 1 + 2 * g
            run_chunk(i1, 1, first=False, last=False)
            run_chunk(i1 + 1, 0, first=False, last=False)
            return carry

        lax.fori_loop(0, (steps - 2) // 2, pair, 0)

        run_chunk(steps - 1, 1, first=False, last=True)
        store_copy(steps - 1, 1).wait()
        feats_giant_copy().wait()

    return fused_kernel


def kernel(x, embed_weight):
    batch, seq, feat = x.shape
    num_rows = batch * seq
    embed_dim = embed_weight.shape[1]

    x2 = x.reshape(num_rows, feat)
    ids = x2[:, 0].astype(jnp.int32).reshape(num_rows // 128, 128)
    feats = x2[:, 1:]
    out = _make_fused(num_rows, embed_dim, feat)(ids, feats, embed_weight)
    return out.reshape(batch, seq, embed_dim + feat - 1)


# speedup vs baseline: 8.0938x; 7.7949x over previous
import functools

import jax
import jax.numpy as jnp
from jax import lax
from jax.experimental import pallas as pl
from jax.experimental.pallas import tpu as pltpu
from jax.experimental.pallas import tpu_sc as plsc

_NC = 2
_NS = 16
_NW = _NC * _NS

_KI = 8
_CHUNK = _KI * 128


def _make_gather(num_rows: int, embed_dim: int):
    assert num_rows % (_NW * _CHUNK) == 0
    rows_per_w = num_rows // _NW
    steps = rows_per_w // _CHUNK
    irows_per_w = rows_per_w // 128
    assert steps >= 4 and steps % 2 == 0

    mesh = plsc.VectorSubcoreMesh(core_axis_name="c", subcore_axis_name="s")

    @functools.partial(
        pl.kernel,
        mesh=mesh,
        out_type=jax.ShapeDtypeStruct((num_rows, embed_dim), jnp.float32),
        scratch_types=[
            pltpu.VMEM((_KI, 128), jnp.int32),
            pltpu.VMEM((_KI, 128), jnp.int32),
            pltpu.VMEM((_CHUNK, 32), jnp.float32),
            pltpu.VMEM((_CHUNK, 32), jnp.float32),
            pltpu.SemaphoreType.DMA,
            pltpu.SemaphoreType.DMA,
            pltpu.SemaphoreType.DMA,
            pltpu.SemaphoreType.DMA,
            pltpu.SemaphoreType.DMA,
            pltpu.SemaphoreType.DMA,
        ],
        compiler_params=pltpu.CompilerParams(use_tc_tiling_on_sc=False),
    )
    def gather_kernel(
        ids_hbm, table_hbm, out_hbm,
        idx0, idx1, emb0, emb1,
        i0, i1, g0, g1, s0, s1,
    ):
        wid = lax.axis_index("s") * _NC + lax.axis_index("c")
        row_base = wid * rows_per_w
        irow_base = wid * irows_per_w
        idx_v = (idx0, idx1)
        emb_v = (emb0, emb1)
        isem = (i0, i1)
        gsem = (g0, g1)
        ssem = (s0, s1)

        def idx_copy(i, b):
            return pltpu.make_async_copy(
                ids_hbm.at[pl.ds(irow_base + i * _KI, _KI)],
                idx_v[b],
                isem[b],
            )

        def gather_copy(b, j):
            return pltpu.make_async_copy(
                table_hbm.at[idx_v[b].at[j]],
                emb_v[b].at[pl.ds(j * 128, 128)],
                gsem[b],
            )

        def store_copy(i, b):
            return pltpu.make_async_copy(
                emb_v[b],
                out_hbm.at[pl.ds(row_base + i * _CHUNK, _CHUNK)],
                ssem[b],
            )

        def run_iter(i, b, head, tail1, tail2):
            if not tail1:
                idx_copy(i + 1, 1 - b).wait()
                if not head:
                    store_copy(i - 1, 1 - b).wait()
                for j in range(_KI):
                    gather_copy(1 - b, j).start()
            for j in range(_KI):
                gather_copy(b, j).wait()
            if not (tail1 or tail2):
                idx_copy(i + 2, b).start()
            store_copy(i, b).start()

        idx_copy(0, 0).start()
        idx_copy(0, 0).wait()
        for j in range(_KI):
            gather_copy(0, j).start()
        idx_copy(1, 1).start()

        run_iter(0, 0, head=True, tail1=False, tail2=False)
        run_iter(1, 1, head=False, tail1=False, tail2=False)

        def pair(g, carry):
            i1 = 2 + 2 * g
            run_iter(i1, 0, head=False, tail1=False, tail2=False)
            run_iter(i1 + 1, 1, head=False, tail1=False, tail2=False)
            return carry

        lax.fori_loop(0, (steps - 4) // 2, pair, 0)

        run_iter(steps - 2, 0, head=False, tail1=False, tail2=True)
        run_iter(steps - 1, 1, head=False, tail1=True, tail2=True)
        store_copy(steps - 2, 0).wait()
        store_copy(steps - 1, 1).wait()

    return gather_kernel


def kernel(x, embed_weight):
    batch, seq, feat = x.shape
    num_rows = batch * seq
    embed_dim = embed_weight.shape[1]

    ids = x[..., 0].astype(jnp.int32).reshape(num_rows // 128, 128)
    emb = _make_gather(num_rows, embed_dim)(ids, embed_weight)
    emb = emb.reshape(batch, seq, embed_dim)
    return jnp.concatenate([emb, x[..., 1:]], axis=-1)
